# baseline (speedup 1.0000x reference)
import functools

import jax
import jax.numpy as jnp
from jax import lax
from jax.experimental import pallas as pl
from jax.experimental.pallas import tpu as pltpu
from jax.experimental.pallas import tpu_sc as plsc

VOCAB = 1000000
D = 128
B = 16384
NC = 2
NS = 16
NW = NC * NS
BPW = B // NW
CH = 64
NCHUNK = BPW // CH
NG = D // 16
NB = CH // 16
ALPHA = 0.02


def _body(node_emb, rel_center, rel_offset, src_hbm, anc_hbm, rid_hbm,
          dist_hbm, inbox_hbm,
          rid_v, cbuf, obuf, idx_s, idx_a, rows_t, rows_a,
          acc_a, acc_b, dist_v, inbox_v,
          sem_t0, sem_a0, sem_t1, sem_a1, sem_t2, sem_a2, sem_out):
    wid = lax.axis_index("s") * NC + lax.axis_index("c")
    base = wid * BPW
    sem_t = [sem_t0, sem_t1, sem_t2]
    sem_a = [sem_a0, sem_a1, sem_a2]

    cp_is = pltpu.async_copy(src_hbm.at[pl.ds(base, BPW)], idx_s, sem_t0)
    cp_ia = pltpu.async_copy(anc_hbm.at[pl.ds(base, BPW)], idx_a, sem_a0)
    cp_is.wait()
    cp_ia.wait()

    def issue(k, sl):
        off = k * CH
        cp_t = pltpu.async_copy(node_emb.at[idx_s.at[pl.ds(off, CH)]],
                                rows_t.at[sl], sem_t[sl])
        cp_a = pltpu.async_copy(node_emb.at[idx_a.at[pl.ds(off, CH)]],
                                rows_a.at[sl], sem_a[sl])
        return cp_t, cp_a

    fifo = [issue(0, 0), issue(1, 1)]

    pltpu.sync_copy(rid_hbm, rid_v)
    cp_c = pltpu.async_copy(rel_center.at[rid_v], cbuf, sem_t2)
    cp_o = pltpu.async_copy(rel_offset.at[rid_v], obuf, sem_a2)
    cp_c.wait()
    cp_o.wait()
    csum = []
    osum = []
    zero = jnp.zeros((16,), jnp.float32)
    for g in range(NG):
        sl = pl.ds(g * 16, 16)
        csum.append(cbuf[0, sl] + cbuf[1, sl])
        osum.append(jnp.maximum(obuf[0, sl], zero) +
                    jnp.maximum(obuf[1, sl], zero))

    out_cps = []

    for k in range(NCHUNK):
        buf = k % 3
        vbuf = k % 2
        if k + 2 < NCHUNK:
            fifo.append(issue(k + 2, (k + 2) % 3))
        cur = fifo.pop(0)
        cur[0].wait()
        cur[1].wait()
        if k >= 2:
            out_cps.pop(0).wait()
            out_cps.pop(0).wait()

        @plsc.parallel_loop(0, CH, unroll=1)
        def row(i):
            s_out = zero
            s_del = zero
            for g in range(NG):
                sl = pl.ds(g * 16, 16)
                delta = jnp.abs(rows_t[buf, i, sl] - rows_a[buf, i, sl]
                                - csum[g])
                s_out = s_out + jnp.maximum(delta - osum[g], zero)
                s_del = s_del + delta
            acc_a[i, :] = s_out
            acc_b[i, :] = s_del

        @plsc.parallel_loop(0, NB, unroll=1)
        def block(b):
            i0 = b * 16
            lane = lax.iota(jnp.int32, 16)
            row_idx = i0 + lane
            s1 = zero
            s2 = zero
            for c in range(16):
                col = jnp.full((16,), c, jnp.int32)
                s1 = s1 + plsc.load_gather(acc_a, [row_idx, col])
                s2 = s2 + plsc.load_gather(acc_b, [row_idx, col])
            dist_v[vbuf, pl.ds(i0, 16)] = (1.0 - ALPHA) * s1 + ALPHA * s2
            inbox_v[vbuf, pl.ds(i0, 16)] = (s1 == 0.0).astype(jnp.int32)

        off = base + k * CH
        out_cps.append(pltpu.async_copy(dist_v.at[vbuf],
                                        dist_hbm.at[pl.ds(off, CH)], sem_out))
        out_cps.append(pltpu.async_copy(inbox_v.at[vbuf],
                                        inbox_hbm.at[pl.ds(off, CH)],
                                        sem_out))

    for cp in out_cps:
        cp.wait()


@jax.jit
def _run(node_emb, rel_center, rel_offset, source_nodes, anchor_nodes,
         rel_ids):
    mesh = plsc.VectorSubcoreMesh(core_axis_name="c", subcore_axis_name="s")
    fn = pl.kernel(
        _body,
        out_type=(
            jax.ShapeDtypeStruct((B,), jnp.float32),
            jax.ShapeDtypeStruct((B,), jnp.int32),
        ),
        mesh=mesh,
        compiler_params=pltpu.CompilerParams(needs_layout_passes=False),
        scratch_types=[
            pltpu.VMEM((2,), jnp.int32),
            pltpu.VMEM((2, D), jnp.float32),
            pltpu.VMEM((2, D), jnp.float32),
            pltpu.VMEM((BPW,), jnp.int32),
            pltpu.VMEM((BPW,), jnp.int32),
            pltpu.VMEM((3, CH, D), jnp.float32),
            pltpu.VMEM((3, CH, D), jnp.float32),
            pltpu.VMEM((CH, 16), jnp.float32),
            pltpu.VMEM((CH, 16), jnp.float32),
            pltpu.VMEM((2, CH), jnp.float32),
            pltpu.VMEM((2, CH), jnp.int32),
            pltpu.SemaphoreType.DMA,
            pltpu.SemaphoreType.DMA,
            pltpu.SemaphoreType.DMA,
            pltpu.SemaphoreType.DMA,
            pltpu.SemaphoreType.DMA,
            pltpu.SemaphoreType.DMA,
            pltpu.SemaphoreType.DMA,
        ],
    )
    return fn(node_emb, rel_center, rel_offset, source_nodes, anchor_nodes,
              rel_ids)


def kernel(node_emb, rel_center, rel_offset, source_nodes, anchor_nodes,
           rel_ids):
    return _run(node_emb, rel_center, rel_offset,
                source_nodes.astype(jnp.int32), anchor_nodes.astype(jnp.int32),
                rel_ids.astype(jnp.int32))


# speedup vs baseline: 1.0507x; 1.0507x over previous
import jax
import jax.numpy as jnp
from jax import lax
from jax.experimental import pallas as pl
from jax.experimental.pallas import tpu as pltpu
from jax.experimental.pallas import tpu_sc as plsc

VOCAB = 1000000
D = 128
B = 16384
NC = 2
NS = 16
NW = NC * NS
BPW = B // NW
CH = 128
NCHUNK = BPW // CH
NG = D // 16
NB = CH // 16
ALPHA = 0.02


def _body(node_emb, rel_center, rel_offset, src_hbm, anc_hbm, rid_hbm,
          dist_hbm, inbox_hbm,
          rid_v, cbuf, obuf, idx_s, idx_a, rows_t, rows_a,
          acc_a, acc_b, dist_v, inbox_v,
          sem_t0, sem_a0, sem_t1, sem_a1, sem_out):
    wid = lax.axis_index("s") * NC + lax.axis_index("c")
    base = wid * BPW
    sem_t = [sem_t0, sem_t1]
    sem_a = [sem_a0, sem_a1]

    cp_is = pltpu.async_copy(src_hbm.at[pl.ds(base, BPW)], idx_s, sem_t0)
    cp_ia = pltpu.async_copy(anc_hbm.at[pl.ds(base, BPW)], idx_a, sem_a0)
    cp_is.wait()
    cp_ia.wait()

    def issue(k, sl):
        off = k * CH
        cp_t = pltpu.async_copy(node_emb.at[idx_s.at[pl.ds(off, CH)]],
                                rows_t.at[sl], sem_t[sl])
        cp_a = pltpu.async_copy(node_emb.at[idx_a.at[pl.ds(off, CH)]],
                                rows_a.at[sl], sem_a[sl])
        return cp_t, cp_a

    pending = issue(0, 0)

    pltpu.sync_copy(rid_hbm, rid_v)
    cp_c = pltpu.async_copy(rel_center.at[rid_v], cbuf, sem_t1)
    cp_o = pltpu.async_copy(rel_offset.at[rid_v], obuf, sem_a1)
    cp_c.wait()
    cp_o.wait()
    csum = []
    osum = []
    zero = jnp.zeros((16,), jnp.float32)
    for g in range(NG):
        sl = pl.ds(g * 16, 16)
        csum.append(cbuf[0, sl] + cbuf[1, sl])
        osum.append(jnp.maximum(obuf[0, sl], zero) +
                    jnp.maximum(obuf[1, sl], zero))

    out_cps = []

    for k in range(NCHUNK):
        buf = k % 2
        if k + 1 < NCHUNK:
            nxt = issue(k + 1, (k + 1) % 2)
        else:
            nxt = None
        pending[0].wait()
        pending[1].wait()
        pending = nxt
        if k >= 2:
            out_cps.pop(0).wait()
            out_cps.pop(0).wait()

        @plsc.parallel_loop(0, CH, unroll=1)
        def row(i):
            s_out = zero
            s_del = zero
            for g in range(NG):
                sl = pl.ds(g * 16, 16)
                delta = jnp.abs(rows_t[buf, i, sl] - rows_a[buf, i, sl]
                                - csum[g])
                s_out = s_out + jnp.maximum(delta - osum[g], zero)
                s_del = s_del + delta
            acc_a[i, :] = s_out
            acc_b[i, :] = s_del

        @plsc.parallel_loop(0, NB, unroll=1)
        def block(b):
            i0 = b * 16
            lane = lax.iota(jnp.int32, 16)
            row_idx = i0 + lane
            s1 = zero
            s2 = zero
            for c in range(16):
                col = jnp.full((16,), c, jnp.int32)
                s1 = s1 + plsc.load_gather(acc_a, [row_idx, col])
                s2 = s2 + plsc.load_gather(acc_b, [row_idx, col])
            dist_v[buf, pl.ds(i0, 16)] = (1.0 - ALPHA) * s1 + ALPHA * s2
            inbox_v[buf, pl.ds(i0, 16)] = (s1 == 0.0).astype(jnp.int32)

        off = base + k * CH
        out_cps.append(pltpu.async_copy(dist_v.at[buf],
                                        dist_hbm.at[pl.ds(off, CH)], sem_out))
        out_cps.append(pltpu.async_copy(inbox_v.at[buf],
                                        inbox_hbm.at[pl.ds(off, CH)],
                                        sem_out))

    for cp in out_cps:
        cp.wait()


@jax.jit
def _run(node_emb, rel_center, rel_offset, source_nodes, anchor_nodes,
         rel_ids):
    mesh = plsc.VectorSubcoreMesh(core_axis_name="c", subcore_axis_name="s")
    fn = pl.kernel(
        _body,
        out_type=(
            jax.ShapeDtypeStruct((B,), jnp.float32),
            jax.ShapeDtypeStruct((B,), jnp.int32),
        ),
        mesh=mesh,
        compiler_params=pltpu.CompilerParams(needs_layout_passes=False),
        scratch_types=[
            pltpu.VMEM((2,), jnp.int32),
            pltpu.VMEM((2, D), jnp.float32),
            pltpu.VMEM((2, D), jnp.float32),
            pltpu.VMEM((BPW,), jnp.int32),
            pltpu.VMEM((BPW,), jnp.int32),
            pltpu.VMEM((2, CH, D), jnp.float32),
            pltpu.VMEM((2, CH, D), jnp.float32),
            pltpu.VMEM((CH, 16), jnp.float32),
            pltpu.VMEM((CH, 16), jnp.float32),
            pltpu.VMEM((2, CH), jnp.float32),
            pltpu.VMEM((2, CH), jnp.int32),
            pltpu.SemaphoreType.DMA,
            pltpu.SemaphoreType.DMA,
            pltpu.SemaphoreType.DMA,
            pltpu.SemaphoreType.DMA,
            pltpu.SemaphoreType.DMA,
        ],
    )
    return fn(node_emb, rel_center, rel_offset, source_nodes, anchor_nodes,
              rel_ids)


def kernel(node_emb, rel_center, rel_offset, source_nodes, anchor_nodes,
           rel_ids):
    return _run(node_emb, rel_center, rel_offset,
                source_nodes.astype(jnp.int32), anchor_nodes.astype(jnp.int32),
                rel_ids.astype(jnp.int32))
